# Initial kernel scaffold; baseline (speedup 1.0000x reference)
import jax
import jax.numpy as jnp
from jax.experimental import pallas as pl


def kernel(x, edge_index, batch, W1, b1, W2, b2, W3, b3):
    raise NotImplementedError("write your pallas kernel here")


# speedup vs baseline: 7.2877x; 7.2877x over previous
import functools

import jax
import jax.numpy as jnp
from jax import lax
from jax.experimental import pallas as pl
from jax.experimental.pallas import tpu as pltpu
from jax.experimental.pallas import tpu_sc as plsc

N = 10000
E = 320000
D = 128
H = 64
C = 10
G = 64

NC = 2
NS = 16
NW = NC * NS
EPW = E // NW
CH = 80
NCHUNK = EPW // CH
NP = 10240
RPT = NP // NS


def _sc_body(x_hbm, z_hbm, src_hbm, dst_hbm, out_hbm,
             src_v, dst_v, rows_v, agg_sh, sem):
    c = lax.axis_index("c")
    s = lax.axis_index("s")
    w = c * NS + s

    pltpu.sync_copy(z_hbm.at[pl.ds(s * RPT, RPT)],
                    agg_sh.at[pl.ds(s * RPT, RPT)])

    pltpu.sync_copy(src_hbm.at[w], src_v)
    pltpu.sync_copy(dst_hbm.at[w], dst_v)
    plsc.subcore_barrier()

    def chunk(j, carry):
        pltpu.async_copy(x_hbm.at[src_v.at[j]], rows_v, sem).wait()
        pltpu.sync_copy(rows_v, agg_sh.at[dst_v.at[j]], add=True)
        return carry

    lax.fori_loop(0, NCHUNK, chunk, 0)
    plsc.subcore_barrier()

    pltpu.sync_copy(agg_sh.at[pl.ds(s * RPT, RPT)],
                    out_hbm.at[c, pl.ds(s * RPT, RPT)])


@functools.cache
def _sc_edge_agg():
    return pl.kernel(
        _sc_body,
        out_type=jax.ShapeDtypeStruct((NC, NP, D), jnp.float32),
        mesh=plsc.VectorSubcoreMesh(core_axis_name="c", subcore_axis_name="s",
                                    num_cores=NC, num_subcores=NS),
        scratch_types=[
            pltpu.VMEM((NCHUNK, CH), jnp.int32),
            pltpu.VMEM((NCHUNK, CH), jnp.int32),
            pltpu.VMEM((CH, D), jnp.float32),
            pltpu.VMEM_SHARED((NP, D), jnp.float32),
            pltpu.SemaphoreType.DMA,
        ],
    )


def _fin_body(x_ref, p_ref, w1_ref, b1_ref, w2_ref, b2_ref, batchT_ref,
              w3_ref, b3_ref, o_ref):
    h = x_ref[...] + p_ref[0, pl.ds(0, N), :] + p_ref[1, pl.ds(0, N), :]
    h1 = jnp.dot(h, w1_ref[...], preferred_element_type=jnp.float32)
    h1 = jnp.maximum(h1 + b1_ref[0][None, :], 0.0)
    h2 = jnp.dot(h1, w2_ref[...], preferred_element_type=jnp.float32)
    h2 = jnp.maximum(h2 + b2_ref[0][None, :], 0.0)
    seg = lax.broadcasted_iota(jnp.int32, (G, N), 0)
    onehotT = (seg == batchT_ref[...]).astype(jnp.float32)
    sums = jnp.dot(onehotT, h2, preferred_element_type=jnp.float32)
    counts = jnp.sum(onehotT, axis=1, keepdims=True)
    pooled = sums / jnp.maximum(counts, 1.0)
    logits = jnp.dot(pooled, w3_ref[...],
                     preferred_element_type=jnp.float32) + b3_ref[0][None, :]
    m = jnp.max(logits, axis=1, keepdims=True)
    lse = jnp.log(jnp.sum(jnp.exp(logits - m), axis=1, keepdims=True)) + m
    o_ref[...] = logits - lse


def _finalize(x, p, W1, b1, W2, b2, batch, W3, b3):
    return pl.pallas_call(
        _fin_body,
        out_shape=jax.ShapeDtypeStruct((G, C), jnp.float32),
    )(x, p, W1, b1.reshape(1, H), W2, b2.reshape(1, H),
      batch.reshape(1, N), W3, b3.reshape(1, C))


def kernel(x, edge_index, batch, W1, b1, W2, b2, W3, b3):
    zeros = jnp.zeros((NP, D), jnp.float32)
    src = edge_index[0].astype(jnp.int32).reshape(NW, NCHUNK, CH)
    dst = edge_index[1].astype(jnp.int32).reshape(NW, NCHUNK, CH)
    p = _sc_edge_agg()(x, zeros, src, dst)
    return _finalize(x, p, W1, b1, W2, b2, batch.astype(jnp.int32), W3, b3)


# speedup vs baseline: 11.3073x; 1.5516x over previous
import functools

import jax
import jax.numpy as jnp
from jax import lax
from jax.experimental import pallas as pl
from jax.experimental.pallas import tpu as pltpu
from jax.experimental.pallas import tpu_sc as plsc

N = 10000
E = 320000
D = 128
H = 64
C = 10
G = 64

NC = 2
NS = 16
NW = NC * NS
EPW = E // NW
CH = 80
NCHUNK = EPW // CH
NBUF = 2
NSTEADY = (NCHUNK - NBUF) // NBUF
NREM = NCHUNK - NBUF - NSTEADY * NBUF
NP = 10240
RPT = NP // NS


def _sc_body(x_hbm, z_hbm, pk_hbm, out_hbm,
             pk_v, src_r, dst_r, rows0, rows1, agg_sh, sem0, sem1):
    c = lax.axis_index("c")
    s = lax.axis_index("s")
    w = c * NS + s
    rows = (rows0, rows1)
    sems = (sem0, sem1)

    pltpu.sync_copy(z_hbm.at[pl.ds(s * RPT, RPT)],
                    agg_sh.at[pl.ds(s * RPT, RPT)])

    pltpu.sync_copy(pk_hbm.at[w], pk_v)
    plsc.subcore_barrier()

    def decode(j, b):
        for k in range(CH // 16):
            v = pk_v[j, pl.ds(k * 16, 16)]
            src_r[b, pl.ds(k * 16, 16)] = jnp.bitwise_and(v, 0xFFFF)
            dst_r[b, pl.ds(k * 16, 16)] = lax.shift_right_logical(v, 16)

    for b in range(NBUF):
        decode(b, b)
        pltpu.async_copy(x_hbm.at[src_r.at[b]], rows[b], sems[b])

    def steady(i, carry):
        j = i * NBUF
        for b in range(NBUF):
            jj = j + b
            pltpu.make_async_copy(x_hbm.at[src_r.at[b]],
                                  rows[b], sems[b]).wait()
            pltpu.sync_copy(rows[b], agg_sh.at[dst_r.at[b]], add=True)
            decode(jj + NBUF, b)
            pltpu.async_copy(x_hbm.at[src_r.at[b]], rows[b], sems[b])
        return carry

    lax.fori_loop(0, NSTEADY, steady, 0)
    for t in range(NBUF + NREM):
        jj = NSTEADY * NBUF + t
        b = t % NBUF
        pltpu.make_async_copy(x_hbm.at[src_r.at[b]],
                              rows[b], sems[b]).wait()
        pltpu.sync_copy(rows[b], agg_sh.at[dst_r.at[b]], add=True)
        if jj + NBUF < NCHUNK:
            decode(jj + NBUF, b)
            pltpu.async_copy(x_hbm.at[src_r.at[b]], rows[b], sems[b])
    plsc.subcore_barrier()

    pltpu.sync_copy(agg_sh.at[pl.ds(s * RPT, RPT)],
                    out_hbm.at[c, pl.ds(s * RPT, RPT)])


@functools.cache
def _sc_edge_agg():
    return pl.kernel(
        _sc_body,
        out_type=jax.ShapeDtypeStruct((NC, NP, D), jnp.float32),
        mesh=plsc.VectorSubcoreMesh(core_axis_name="c", subcore_axis_name="s",
                                    num_cores=NC, num_subcores=NS),
        scratch_types=[
            pltpu.VMEM((NCHUNK, CH), jnp.int32),
            pltpu.VMEM((NBUF, CH), jnp.int32),
            pltpu.VMEM((NBUF, CH), jnp.int32),
            pltpu.VMEM((CH, D), jnp.float32),
            pltpu.VMEM((CH, D), jnp.float32),
            pltpu.VMEM_SHARED((NP, D), jnp.float32),
            pltpu.SemaphoreType.DMA,
            pltpu.SemaphoreType.DMA,
        ],
    )


def _fin_body(x_ref, p_ref, w1_ref, b1_ref, w2_ref, b2_ref, batchT_ref,
              w3_ref, b3_ref, o_ref):
    h = x_ref[...] + p_ref[0, pl.ds(0, N), :] + p_ref[1, pl.ds(0, N), :]
    h1 = jnp.dot(h, w1_ref[...], preferred_element_type=jnp.float32)
    h1 = jnp.maximum(h1 + b1_ref[0][None, :], 0.0)
    h2 = jnp.dot(h1, w2_ref[...], preferred_element_type=jnp.float32)
    h2 = jnp.maximum(h2 + b2_ref[0][None, :], 0.0)
    seg = lax.broadcasted_iota(jnp.int32, (G, N), 0)
    onehotT = (seg == batchT_ref[...]).astype(jnp.float32)
    sums = jnp.dot(onehotT, h2, preferred_element_type=jnp.float32)
    counts = jnp.sum(onehotT, axis=1, keepdims=True)
    pooled = sums / jnp.maximum(counts, 1.0)
    logits = jnp.dot(pooled, w3_ref[...],
                     preferred_element_type=jnp.float32) + b3_ref[0][None, :]
    m = jnp.max(logits, axis=1, keepdims=True)
    lse = jnp.log(jnp.sum(jnp.exp(logits - m), axis=1, keepdims=True)) + m
    o_ref[...] = logits - lse


def _finalize(x, p, W1, b1, W2, b2, batch, W3, b3):
    return pl.pallas_call(
        _fin_body,
        out_shape=jax.ShapeDtypeStruct((G, C), jnp.float32),
    )(x, p, W1, b1.reshape(1, H), W2, b2.reshape(1, H),
      batch.reshape(1, N), W3, b3.reshape(1, C))


def kernel(x, edge_index, batch, W1, b1, W2, b2, W3, b3):
    zeros = jnp.zeros((NP, D), jnp.float32)
    src = edge_index[0].astype(jnp.int32)
    dst = edge_index[1].astype(jnp.int32)
    packed = (src | (dst << 16)).reshape(NW, NCHUNK, CH)
    p = _sc_edge_agg()(x, zeros, packed)
    return _finalize(x, p, W1, b1, W2, b2, batch.astype(jnp.int32), W3, b3)
